# baseline (speedup 1.0000x reference)
import functools

import jax
import jax.numpy as jnp
from jax import lax
from jax.experimental import pallas as pl
from jax.experimental.pallas import tpu as pltpu
from jax.experimental.pallas import tpu_sc as plsc

_K = 50
_B = 128
_N = 32768
_L = 16
_NVEC = _N // _L
_BSHIFT = 24
_NBKT = 1 << (31 - _BSHIFT)
_CAP = _NVEC
_ROWS_PER_W = 4
_FAST_VECS = 16
_INF_BITS = 0x7F800000


def _row_bottom_k(base, row2_v, cand_v, counts_v, totals_v):
    lanes = lax.iota(jnp.int32, _L)
    ones_i = jnp.ones((_L,), jnp.int32)
    zeros_i = jnp.zeros((_L,), jnp.int32)
    zeros_f = jnp.zeros((_L,), jnp.float32)
    lane_cap = lanes * _CAP

    @plsc.parallel_loop(0, _NVEC, unroll=16)
    def _hist(j):
        x = row2_v[pl.ds(base + j * _L, _L)]
        a = lax.bitcast_convert_type(jnp.abs(x), jnp.int32)
        e = lax.shift_right_logical(a, _BSHIFT)
        idx = e * _L + lanes
        plsc.addupdate_scatter(counts_v, [idx], ones_i)

    lane0 = lanes == 0

    @plsc.parallel_loop(0, _NBKT, unroll=8)
    def _tot(b):
        s = jnp.sum(counts_v[pl.ds(b * _L, _L)])
        counts_v[pl.ds(b * _L, _L)] = zeros_i
        bvec = jnp.full((_L,), b, jnp.int32)
        plsc.store_scatter(totals_v, [bvec], jnp.full((_L,), s, jnp.int32),
                           mask=lane0)

    def _scan(g, carry):
        cum, found, m = carry
        v = totals_v[pl.ds(g * _L, _L)]
        cs = plsc.cumsum(v) + cum
        hit = cs >= _K
        any_hit = jnp.any(hit)
        f = jnp.max(plsc.all_reduce_ffs(hit))
        take = jnp.logical_and(any_hit, jnp.logical_not(found))
        m = jnp.where(take, g * _L + f, m)
        found = jnp.logical_or(found, any_hit)
        cum = cum + jnp.sum(v)
        return cum, found, m

    _, _, m = lax.fori_loop(
        0, _NBKT // _L, _scan, (jnp.int32(0), False, jnp.int32(0)))

    m_hi = lax.shift_left(m + 1, _BSHIFT)

    def _compact(j, off):
        x = row2_v[pl.ds(base + j * _L, _L)]
        xa = jnp.abs(x)
        a = lax.bitcast_convert_type(xa, jnp.int32)
        le = a < m_hi
        idx = lane_cap + off
        plsc.store_scatter(cand_v, [idx], xa, mask=le)
        return off + le.astype(jnp.int32)

    off = plsc.parallel_loop(0, _NVEC, unroll=8, carry=zeros_i)(_compact)
    lmax = jnp.max(off)
    r = jnp.int32(_K)

    prefix0 = lax.shift_left(m, _BSHIFT)

    prefix0_v = jnp.full((_L,), 1, jnp.int32) * prefix0

    def _fast(off, r, prefix):
        cbs = []
        for j in range(_FAST_VECS):
            c = plsc.load_gather(cand_v, [lane_cap + j])
            cb = lax.bitcast_convert_type(c, jnp.int32)
            cbs.append(jnp.where(off > j, cb, jnp.int32(_INF_BITS)))
        for bit in range(_BSHIFT - 1, -1, -1):
            t = prefix | (1 << bit)
            cnt = zeros_i
            for cb in cbs:
                cnt = cnt + plsc.all_reduce_population_count(cb < t)
            prefix = jnp.where(cnt >= r, prefix, t)
        s_lt = zeros_f
        c_lt = zeros_i
        for cb in cbs:
            lt = cb < prefix
            s_lt = s_lt + jnp.where(
                lt, lax.bitcast_convert_type(cb, jnp.float32), zeros_f)
            c_lt = c_lt + plsc.all_reduce_population_count(lt)
        return jnp.sum(s_lt), c_lt, prefix

    def _slow(off, r, prefix):
        lm = jnp.max(off)

        def _bit(ii, prefix):
            t = prefix | lax.shift_left(jnp.int32(1), _BSHIFT - 1 - ii)

            def _cnt(j, cnt):
                c = plsc.load_gather(cand_v, [lane_cap + j])
                cb = lax.bitcast_convert_type(c, jnp.int32)
                ok = jnp.logical_and(j < off, cb < t)
                return cnt + plsc.all_reduce_population_count(ok)

            cnt = lax.fori_loop(0, lm, _cnt, zeros_i)
            return jnp.where(cnt >= r, prefix, t)

        prefix = lax.fori_loop(0, _BSHIFT, _bit, prefix)

        def _fin(j, carry):
            s_lt, c_lt = carry
            c = plsc.load_gather(cand_v, [lane_cap + j])
            cb = lax.bitcast_convert_type(c, jnp.int32)
            ok = jnp.logical_and(j < off, cb < prefix)
            s_lt = s_lt + jnp.where(ok, c, zeros_f)
            c_lt = c_lt + plsc.all_reduce_population_count(ok)
            return s_lt, c_lt

        s_lt, c_lt = lax.fori_loop(0, lm, _fin, (zeros_f, zeros_i))
        return jnp.sum(s_lt), c_lt, prefix

    s_lt, c_lt, prefix = lax.cond(
        lmax <= _FAST_VECS, _fast, _slow, off, r, prefix0_v)

    t_val = lax.bitcast_convert_type(prefix, jnp.float32)
    ties = (r - c_lt).astype(jnp.float32)
    return s_lt + ties * t_val


@functools.partial(
    pl.kernel,
    out_type=jax.ShapeDtypeStruct((_B // _ROWS_PER_W, _L), jnp.float32),
    mesh=plsc.VectorSubcoreMesh(core_axis_name="c", subcore_axis_name="s"),
    compiler_params=pltpu.CompilerParams(
        needs_layout_passes=False,
        skip_device_barrier=True,
        disable_bounds_checks=True,
        disable_semaphore_checks=True,
    ),
    scratch_types=[
        pltpu.VMEM((2 * _N,), jnp.float32),
        pltpu.VMEM((_N,), jnp.float32),
        pltpu.VMEM((_NBKT * _L,), jnp.int32),
        pltpu.VMEM((_NBKT,), jnp.int32),
        pltpu.VMEM((_L,), jnp.float32),
        pltpu.SemaphoreType.DMA,
        pltpu.SemaphoreType.DMA,
    ],
)
def _sc_bottom_k(scores_hbm, out_hbm, row2_v, cand_v, counts_v,
                 totals_v, stage_v, sem_a, sem_b):
    wid = lax.axis_index("s") * 2 + lax.axis_index("c")
    row0 = wid * _ROWS_PER_W

    @plsc.parallel_loop(0, _NBKT, unroll=8)
    def _zero(b):
        counts_v[pl.ds(b * _L, _L)] = jnp.zeros((_L,), jnp.int32)

    pltpu.async_copy(scores_hbm.at[row0], row2_v.at[pl.ds(0, _N)], sem_a)
    pltpu.async_copy(scores_hbm.at[row0 + 1], row2_v.at[pl.ds(_N, _N)],
                     sem_b)

    lanes = lax.iota(jnp.int32, _L)

    def _row(i, acc):
        row = row0 + i
        parity = i & 1
        base = parity * _N

        @pl.when(parity == 0)
        def _wait_a():
            pltpu.make_async_copy(
                scores_hbm.at[row], row2_v.at[pl.ds(0, _N)], sem_a).wait()

        @pl.when(parity == 1)
        def _wait_b():
            pltpu.make_async_copy(
                scores_hbm.at[row], row2_v.at[pl.ds(_N, _N)], sem_b).wait()

        row_sum = _row_bottom_k(base, row2_v, cand_v, counts_v, totals_v)

        @pl.when(jnp.logical_and(i < _ROWS_PER_W - 2, parity == 0))
        def _pre_a():
            pltpu.async_copy(scores_hbm.at[row + 2],
                             row2_v.at[pl.ds(0, _N)], sem_a)

        @pl.when(jnp.logical_and(i < _ROWS_PER_W - 2, parity == 1))
        def _pre_b():
            pltpu.async_copy(scores_hbm.at[row + 2],
                             row2_v.at[pl.ds(_N, _N)], sem_b)

        return jnp.where(lanes == i, row_sum, acc)

    acc = lax.fori_loop(0, _ROWS_PER_W, _row, jnp.zeros((_L,), jnp.float32))
    stage_v[...] = acc
    pltpu.sync_copy(stage_v, out_hbm.at[wid])


def kernel(scores):
    out = _sc_bottom_k(scores)
    return jnp.mean(out[:, :_ROWS_PER_W])


# speedup vs baseline: 1.3235x; 1.3235x over previous
import functools

import jax
import jax.numpy as jnp
from jax import lax
from jax.experimental import pallas as pl
from jax.experimental.pallas import tpu as pltpu
from jax.experimental.pallas import tpu_sc as plsc

_K = 50
_B = 128
_N = 32768
_L = 16
_NVEC = _N // _L
_BSHIFT = 23
_NBKT = 1 << (31 - _BSHIFT)
_CAP = _NVEC
_ROWS_PER_W = 4
_FAST_VECS = 12
_INF_BITS = 0x7F800000


def _row_bottom_k(base, row2_v, cand_v, counts_v, totals_v):
    lanes = lax.iota(jnp.int32, _L)
    ones_i = jnp.ones((_L,), jnp.int32)
    zeros_i = jnp.zeros((_L,), jnp.int32)
    zeros_f = jnp.zeros((_L,), jnp.float32)
    lane_cap = lanes * _CAP

    @plsc.parallel_loop(0, _NVEC, unroll=16)
    def _hist(j):
        x = row2_v[pl.ds(base + j * _L, _L)]
        a = lax.bitcast_convert_type(jnp.abs(x), jnp.int32)
        e = lax.shift_right_logical(a, _BSHIFT)
        idx = e * _L + lanes
        plsc.addupdate_scatter(counts_v, [idx], ones_i)

    lane0 = lanes == 0

    @plsc.parallel_loop(0, _NBKT, unroll=8)
    def _tot(b):
        s = jnp.sum(counts_v[pl.ds(b * _L, _L)])
        counts_v[pl.ds(b * _L, _L)] = zeros_i
        bvec = jnp.full((_L,), b, jnp.int32)
        plsc.store_scatter(totals_v, [bvec], jnp.full((_L,), s, jnp.int32),
                           mask=lane0)

    def _scan(g, carry):
        cum, found, m = carry
        v = totals_v[pl.ds(g * _L, _L)]
        cs = plsc.cumsum(v) + cum
        hit = cs >= _K
        any_hit = jnp.any(hit)
        f = jnp.max(plsc.all_reduce_ffs(hit))
        take = jnp.logical_and(any_hit, jnp.logical_not(found))
        m = jnp.where(take, g * _L + f, m)
        found = jnp.logical_or(found, any_hit)
        cum = cum + jnp.sum(v)
        return cum, found, m

    _, _, m = lax.fori_loop(
        0, _NBKT // _L, _scan, (jnp.int32(0), False, jnp.int32(0)))

    m_hi = lax.shift_left(m + 1, _BSHIFT)

    def _compact(j, off):
        x = row2_v[pl.ds(base + j * _L, _L)]
        xa = jnp.abs(x)
        a = lax.bitcast_convert_type(xa, jnp.int32)
        le = a < m_hi
        idx = lane_cap + off
        plsc.store_scatter(cand_v, [idx], xa, mask=le)
        return off + le.astype(jnp.int32)

    off = plsc.parallel_loop(0, _NVEC, unroll=8, carry=zeros_i)(_compact)
    lmax = jnp.max(off)
    r = jnp.int32(_K)

    prefix0 = lax.shift_left(m, _BSHIFT)

    prefix0_v = jnp.full((_L,), 1, jnp.int32) * prefix0

    def _fast(off, r, prefix):
        cbs = []
        for j in range(_FAST_VECS):
            c = plsc.load_gather(cand_v, [lane_cap + j])
            cb = lax.bitcast_convert_type(c, jnp.int32)
            cbs.append(jnp.where(off > j, cb, jnp.int32(_INF_BITS)))
        for bit in range(_BSHIFT - 1, -1, -1):
            t = prefix | (1 << bit)
            cnt = zeros_i
            for cb in cbs:
                cnt = cnt + plsc.all_reduce_population_count(cb < t)
            prefix = jnp.where(cnt >= r, prefix, t)
        s_lt = zeros_f
        c_lt = zeros_i
        for cb in cbs:
            lt = cb < prefix
            s_lt = s_lt + jnp.where(
                lt, lax.bitcast_convert_type(cb, jnp.float32), zeros_f)
            c_lt = c_lt + plsc.all_reduce_population_count(lt)
        return jnp.sum(s_lt), c_lt, prefix

    def _slow(off, r, prefix):
        lm = jnp.max(off)

        def _bit(ii, prefix):
            t = prefix | lax.shift_left(jnp.int32(1), _BSHIFT - 1 - ii)

            def _cnt(j, cnt):
                c = plsc.load_gather(cand_v, [lane_cap + j])
                cb = lax.bitcast_convert_type(c, jnp.int32)
                ok = jnp.logical_and(j < off, cb < t)
                return cnt + plsc.all_reduce_population_count(ok)

            cnt = lax.fori_loop(0, lm, _cnt, zeros_i)
            return jnp.where(cnt >= r, prefix, t)

        prefix = lax.fori_loop(0, _BSHIFT, _bit, prefix)

        def _fin(j, carry):
            s_lt, c_lt = carry
            c = plsc.load_gather(cand_v, [lane_cap + j])
            cb = lax.bitcast_convert_type(c, jnp.int32)
            ok = jnp.logical_and(j < off, cb < prefix)
            s_lt = s_lt + jnp.where(ok, c, zeros_f)
            c_lt = c_lt + plsc.all_reduce_population_count(ok)
            return s_lt, c_lt

        s_lt, c_lt = lax.fori_loop(0, lm, _fin, (zeros_f, zeros_i))
        return jnp.sum(s_lt), c_lt, prefix

    s_lt, c_lt, prefix = lax.cond(
        lmax <= _FAST_VECS, _fast, _slow, off, r, prefix0_v)

    t_val = lax.bitcast_convert_type(prefix, jnp.float32)
    ties = (r - c_lt).astype(jnp.float32)
    return s_lt + ties * t_val


@functools.partial(
    pl.kernel,
    out_type=jax.ShapeDtypeStruct((_B // _ROWS_PER_W, _L), jnp.float32),
    mesh=plsc.VectorSubcoreMesh(core_axis_name="c", subcore_axis_name="s"),
    compiler_params=pltpu.CompilerParams(
        needs_layout_passes=False,
        skip_device_barrier=True,
        disable_bounds_checks=True,
        disable_semaphore_checks=True,
    ),
    scratch_types=[
        pltpu.VMEM((2 * _N,), jnp.float32),
        pltpu.VMEM((_N,), jnp.float32),
        pltpu.VMEM((_NBKT * _L,), jnp.int32),
        pltpu.VMEM((_NBKT,), jnp.int32),
        pltpu.VMEM((_L,), jnp.float32),
        pltpu.SemaphoreType.DMA,
        pltpu.SemaphoreType.DMA,
    ],
)
def _sc_bottom_k(scores_hbm, out_hbm, row2_v, cand_v, counts_v,
                 totals_v, stage_v, sem_a, sem_b):
    wid = lax.axis_index("s") * 2 + lax.axis_index("c")
    row0 = wid * _ROWS_PER_W

    @plsc.parallel_loop(0, _NBKT, unroll=8)
    def _zero(b):
        counts_v[pl.ds(b * _L, _L)] = jnp.zeros((_L,), jnp.int32)

    pltpu.async_copy(scores_hbm.at[row0], row2_v.at[pl.ds(0, _N)], sem_a)
    pltpu.async_copy(scores_hbm.at[row0 + 1], row2_v.at[pl.ds(_N, _N)],
                     sem_b)

    lanes = lax.iota(jnp.int32, _L)

    def _row(i, acc):
        row = row0 + i
        parity = i & 1
        base = parity * _N

        @pl.when(parity == 0)
        def _wait_a():
            pltpu.make_async_copy(
                scores_hbm.at[row], row2_v.at[pl.ds(0, _N)], sem_a).wait()

        @pl.when(parity == 1)
        def _wait_b():
            pltpu.make_async_copy(
                scores_hbm.at[row], row2_v.at[pl.ds(_N, _N)], sem_b).wait()

        row_sum = _row_bottom_k(base, row2_v, cand_v, counts_v, totals_v)

        @pl.when(jnp.logical_and(i < _ROWS_PER_W - 2, parity == 0))
        def _pre_a():
            pltpu.async_copy(scores_hbm.at[row + 2],
                             row2_v.at[pl.ds(0, _N)], sem_a)

        @pl.when(jnp.logical_and(i < _ROWS_PER_W - 2, parity == 1))
        def _pre_b():
            pltpu.async_copy(scores_hbm.at[row + 2],
                             row2_v.at[pl.ds(_N, _N)], sem_b)

        return jnp.where(lanes == i, row_sum, acc)

    acc = lax.fori_loop(0, _ROWS_PER_W, _row, jnp.zeros((_L,), jnp.float32))
    stage_v[...] = acc
    pltpu.sync_copy(stage_v, out_hbm.at[wid])


def kernel(scores):
    out = _sc_bottom_k(scores)
    return jnp.mean(out[:, :_ROWS_PER_W])
